# baseline (speedup 1.0000x reference)
import functools

import jax
import jax.numpy as jnp
from jax import lax
from jax.experimental import pallas as pl
from jax.experimental.pallas import tpu as pltpu
from jax.experimental.pallas import tpu_sc as plsc

BATCH = 16384
D = 64
V = 1000000
NUM_CORES = 2
NUM_SUBCORES = 16
NUM_WORKERS = NUM_CORES * NUM_SUBCORES
B_PER_W = BATCH // NUM_WORKERS
LANES = 16
CH = 512
VR = 31232
TAIL_SEL = 999936
TAIL_WIN = 999872
SENTINEL = 0x3FFFFFFF
E_ROWS = BATCH + LANES

_mesh = plsc.VectorSubcoreMesh(core_axis_name="c", subcore_axis_name="s")


def _wid():
    return lax.axis_index("s") * NUM_CORES + lax.axis_index("c")


@functools.partial(
    pl.kernel,
    mesh=_mesh,
    out_type=jax.ShapeDtypeStruct((E_ROWS, 2 * D), jnp.float32),
    scratch_types=[
        pltpu.VMEM((BATCH + LANES,), jnp.int32),
        pltpu.VMEM((BATCH + LANES,), jnp.int32),
        pltpu.VMEM((BATCH + LANES,), jnp.int32),
        pltpu.VMEM((D, CH), jnp.float32),
        pltpu.VMEM((D, 2 * D), jnp.float32),
        pltpu.VMEM((D, LANES), jnp.float32),
        pltpu.VMEM((4, LANES, 2 * D), jnp.float32),
        pltpu.VMEM((4, LANES), jnp.int32),
        pltpu.SMEM((2,), jnp.int32),
        pltpu.SemaphoreType.DMA,
        pltpu.SemaphoreType.DMA,
    ],
    compiler_params=pltpu.CompilerParams(
        needs_layout_passes=False, disable_bounds_checks=True),
)
def _extract(idx_hbm, tT_hbm, tail_hbm, e_hbm,
             idx_v, hb_v, cb_v, chunk_v, tail_v, trans_v, ring_v, bidx_v,
             outs_s, sem_c, sem_r):
    wid = _wid()
    lane16 = lax.iota(jnp.int32, LANES)

    pltpu.sync_copy(idx_hbm, idx_v.at[pl.ds(0, BATCH)])
    pltpu.sync_copy(tail_hbm, tail_v)
    idx_v[pl.ds(BATCH, LANES)] = jnp.full((LANES,), SENTINEL, jnp.int32)
    outs_s[0] = 0
    outs_s[1] = 0

    v_lo = wid * VR
    sel_hi = jnp.where(wid == NUM_WORKERS - 1, V, v_lo + VR)

    def coll(j, cnt):
        u = idx_v[pl.ds(pl.multiple_of(j * LANES, LANES), LANES)]
        b = j * LANES + lane16
        msk = (u >= v_lo) & (u < sel_hi)
        pref = plsc.cumsum(jnp.where(msk, 1, 0))
        plsc.store_scatter(hb_v, [cnt + pref - 1], b, mask=msk)
        return cnt + pref[LANES - 1]

    cnt = lax.fori_loop(0, BATCH // LANES, coll, jnp.int32(0))
    plsc.store_scatter(hb_v, [cnt + lane16],
                       jnp.full((LANES,), BATCH, jnp.int32),
                       mask=lane16 >= 0)

    def process(buf, width, win_lo, sel_lo, sel_hi_):
        def scan(h, ccnt):
            hb16 = hb_v[pl.ds(pl.multiple_of(h * LANES, LANES), LANES)]
            hu = plsc.load_gather(idx_v, [hb16])
            msk = (hu >= sel_lo) & (hu < sel_hi_)
            pref = plsc.cumsum(jnp.where(msk, 1, 0))
            plsc.store_scatter(cb_v, [ccnt + pref - 1], hb16, mask=msk)
            return ccnt + pref[LANES - 1]

        nh = (cnt + LANES) >> 4
        ccnt = lax.fori_loop(0, nh, scan, jnp.int32(0))
        plsc.store_scatter(cb_v, [ccnt + lane16],
                           jnp.full((LANES,), BATCH, jnp.int32),
                           mask=lane16 >= 0)

        def ext(h, carry):
            cb16 = cb_v[pl.ds(pl.multiple_of(h * LANES, LANES), LANES)]
            hu = plsc.load_gather(idx_v, [cb16])
            cols = jnp.clip(hu - win_lo, 0, width - 1)
            for d in range(D):
                trans_v[d, pl.ds(0, LANES)] = plsc.load_gather(
                    buf, [jnp.full((LANES,), d, jnp.int32), cols])
            slot = outs_s[1] & 3
            outs_s[1] = outs_s[1] + 1

            @pl.when(outs_s[0] >= 4)
            def _():
                pltpu.make_async_copy(
                    e_hbm.at[pl.ds(BATCH, LANES)],
                    ring_v.at[slot], sem_r).wait()
                outs_s[0] = outs_s[0] - 1

            for hl in range(LANES):
                for k in range(D // LANES):
                    dv = k * LANES + lane16
                    ring_v[slot, hl, pl.ds(k * LANES, LANES)] = (
                        plsc.load_gather(
                            trans_v,
                            [dv, jnp.full((LANES,), hl, jnp.int32)]))
            bidx_v[slot, pl.ds(0, LANES)] = cb16
            pltpu.async_copy(ring_v.at[slot], e_hbm.at[bidx_v.at[slot]],
                             sem_r)
            outs_s[0] = outs_s[0] + 1
            return carry

        nc = (ccnt + LANES) >> 4
        lax.fori_loop(0, nc, ext, jnp.int32(0))

    n_chunks = jnp.where(wid == NUM_WORKERS - 1,
                         (VR + CH) // CH, VR // CH)

    def chunk_body(c, carry):
        v0 = pl.multiple_of(v_lo + c * CH, CH)
        pltpu.async_copy(
            tT_hbm.at[pl.ds(0, D), pl.ds(v0, CH)], chunk_v, sem_c).wait()
        process(chunk_v, CH, v0, v0, jnp.minimum(v0 + CH, TAIL_SEL))
        return carry

    lax.fori_loop(0, n_chunks, chunk_body, jnp.int32(0))

    @pl.when(wid == NUM_WORKERS - 1)
    def _():
        process(tail_v, 2 * D, TAIL_WIN, TAIL_SEL, V)

    for i in range(4):
        @pl.when(outs_s[0] > i)
        def _():
            pltpu.make_async_copy(
                e_hbm.at[pl.ds(BATCH, LANES)], ring_v.at[i], sem_r).wait()


@functools.partial(
    pl.kernel,
    mesh=_mesh,
    out_type=jax.ShapeDtypeStruct((BATCH,), jnp.float32),
    scratch_types=[
        pltpu.VMEM((2 * D, 2 * D), jnp.float32),
        pltpu.VMEM((2 * D, 2 * D), jnp.float32),
        pltpu.VMEM((B_PER_W,), jnp.float32),
        pltpu.SemaphoreType.DMA,
    ],
    compiler_params=pltpu.CompilerParams(
        needs_layout_passes=False, disable_bounds_checks=True),
)
def _dot(eu_hbm, em_hbm, out_hbm, ub, mb, out_v, sem):
    wid = _wid()
    base = wid * B_PER_W
    lane16 = lax.iota(jnp.int32, LANES)

    for c in range(B_PER_W // (2 * D)):
        pltpu.sync_copy(
            eu_hbm.at[pl.ds(base + c * 2 * D, 2 * D), pl.ds(0, 2 * D)], ub)
        pltpu.sync_copy(
            em_hbm.at[pl.ds(base + c * 2 * D, 2 * D), pl.ds(0, 2 * D)], mb)

        def group(g, carry):
            rows = g * LANES + lane16
            acc = jnp.zeros((LANES,), jnp.float32)
            for d in range(D):
                dv = jnp.full((LANES,), d, jnp.int32)
                acc = acc + (plsc.load_gather(ub, [rows, dv])
                             * plsc.load_gather(mb, [rows, dv]))
            out_v[pl.ds(pl.multiple_of(c * 2 * D + g * LANES, LANES),
                        LANES)] = acc
            return carry

        lax.fori_loop(0, 2 * D // LANES, group, 0)

    pltpu.sync_copy(out_v, out_hbm.at[pl.ds(base, B_PER_W)])


def kernel(userIndices, movieIndices, user_table, movie_table):
    ui = userIndices.astype(jnp.int32)
    mi = movieIndices.astype(jnp.int32)
    utT = user_table.T
    mtT = movie_table.T
    utail = lax.slice(utT, (0, TAIL_WIN), (D, V))
    mtail = lax.slice(mtT, (0, TAIL_WIN), (D, V))
    eu = _extract(ui, utT, utail)
    em = _extract(mi, mtT, mtail)
    return _dot(eu, em)


# speedup vs baseline: 3.9085x; 3.9085x over previous
import functools

import jax
import jax.numpy as jnp
from jax import lax
from jax.experimental import pallas as pl
from jax.experimental.pallas import tpu as pltpu
from jax.experimental.pallas import tpu_sc as plsc

BATCH = 16384
D = 64
NUM_CORES = 2
NUM_SUBCORES = 16
NUM_WORKERS = NUM_CORES * NUM_SUBCORES
B_PER_W = BATCH // NUM_WORKERS
LANES = 16
WIN = 128
N_SUB = B_PER_W // 2

_mesh = plsc.VectorSubcoreMesh(core_axis_name="c", subcore_axis_name="s")


@functools.partial(
    pl.kernel,
    mesh=_mesh,
    out_type=jax.ShapeDtypeStruct((BATCH,), jnp.float32),
    scratch_types=[
        pltpu.VMEM((B_PER_W + LANES,), jnp.int32),
        pltpu.VMEM((B_PER_W + LANES,), jnp.int32),
        pltpu.VMEM((D, 2 * WIN), jnp.float32),
        pltpu.VMEM((D, 2 * WIN), jnp.float32),
        pltpu.VMEM((D, 2 * WIN), jnp.float32),
        pltpu.VMEM((D, 2 * WIN), jnp.float32),
        pltpu.VMEM((D, 2 * WIN), jnp.float32),
        pltpu.VMEM((D, 2 * WIN), jnp.float32),
        pltpu.VMEM((LANES,), jnp.float32),
        pltpu.VMEM((B_PER_W,), jnp.float32),
        pltpu.SemaphoreType.DMA,
        pltpu.SemaphoreType.DMA,
        pltpu.SemaphoreType.DMA,
    ],
    compiler_params=pltpu.CompilerParams(
        needs_layout_passes=False, disable_bounds_checks=True),
)
def _sc_dot(uidx_hbm, midx_hbm, utT_hbm, mtT_hbm, out_hbm,
            uidx_v, midx_v, ubufA, mbufA, ubufB, mbufB, ubufC, mbufC,
            shuf, out_v, semA, semB, semC):
    wid = lax.axis_index("s") * NUM_CORES + lax.axis_index("c")
    base = wid * B_PER_W

    pltpu.sync_copy(uidx_hbm.at[wid], uidx_v.at[pl.ds(0, B_PER_W)])
    pltpu.sync_copy(midx_hbm.at[wid], midx_v.at[pl.ds(0, B_PER_W)])

    lane16 = lax.iota(jnp.int32, LANES)
    oct_id = lane16 & 7
    half_id = lane16 >> 3

    def idx_pair(s):
        uv = uidx_v[pl.ds(2 * s, LANES)]
        mv = midx_v[pl.ds(2 * s, LANES)]
        return uv[0], uv[1], mv[0], mv[1]

    def issue(s, ubuf, mbuf, sem):
        u0, u1, m0, m1 = idx_pair(s)
        cps = []
        for k, (u, m) in enumerate(((u0, m0), (u1, m1))):
            su = pl.multiple_of((u >> 7) << 7, WIN)
            sm = pl.multiple_of((m >> 7) << 7, WIN)
            cps.append(pltpu.async_copy(
                utT_hbm.at[pl.ds(0, D), pl.ds(su, WIN)],
                ubuf.at[pl.ds(0, D), pl.ds(k * WIN, WIN)], sem))
            cps.append(pltpu.async_copy(
                mtT_hbm.at[pl.ds(0, D), pl.ds(sm, WIN)],
                mbuf.at[pl.ds(0, D), pl.ds(k * WIN, WIN)], sem))
        return cps

    def compute(s, ubuf, mbuf, acc):
        acc = jnp.where((s & 7) == 0, jnp.zeros((LANES,), jnp.float32), acc)
        u0, u1, m0, m1 = idx_pair(s)
        ucol = (half_id << 7) + jnp.where(
            lane16 < 8, u0 & (WIN - 1), u1 & (WIN - 1))
        mcol = (half_id << 7) + jnp.where(
            lane16 < 8, m0 & (WIN - 1), m1 & (WIN - 1))
        acc2 = jnp.zeros((LANES,), jnp.float32)
        for d0 in range(0, D, 8):
            dvec = d0 + oct_id
            uv = plsc.load_gather(ubuf, [dvec, ucol])
            mv = plsc.load_gather(mbuf, [dvec, mcol])
            acc2 = acc2 + uv * mv
        for step in (1, 2, 4):
            shuf[...] = acc2
            acc2 = acc2 + plsc.load_gather(shuf, [lane16 ^ step])
        p0 = (s & 7) * 2
        acc = jnp.where(lane16 == p0, acc2[0], acc)
        acc = jnp.where(lane16 == p0 + 1, acc2[8], acc)
        out_v[pl.ds(pl.multiple_of((s >> 3) * LANES, LANES), LANES)] = acc
        return acc

    def body(i, acc):
        s0 = 3 * i
        cpsA = issue(s0, ubufA, mbufA, semA)
        cpsB = issue(s0 + 1, ubufB, mbufB, semB)
        cpsC = issue(s0 + 2, ubufC, mbufC, semC)
        for cp in cpsA:
            cp.wait()
        acc = compute(s0, ubufA, mbufA, acc)
        for cp in cpsB:
            cp.wait()
        acc = compute(s0 + 1, ubufB, mbufB, acc)
        for cp in cpsC:
            cp.wait()
        acc = compute(s0 + 2, ubufC, mbufC, acc)
        return acc

    accf = lax.fori_loop(0, N_SUB // 3, body,
                         jnp.zeros((LANES,), jnp.float32))
    last = N_SUB - (N_SUB % 3)
    for cp in issue(last, ubufA, mbufA, semA):
        cp.wait()
    compute(last, ubufA, mbufA, accf)

    pltpu.sync_copy(out_v, out_hbm.at[pl.ds(base, B_PER_W)])


def kernel(userIndices, movieIndices, user_table, movie_table):
    u = userIndices.astype(jnp.int32).reshape(NUM_WORKERS, B_PER_W)
    m = movieIndices.astype(jnp.int32).reshape(NUM_WORKERS, B_PER_W)
    return _sc_dot(u, m, user_table.T, movie_table.T)
